# Initial kernel scaffold; baseline (speedup 1.0000x reference)
import jax
import jax.numpy as jnp
from jax.experimental import pallas as pl


def kernel(x, edge_index, edge_attr, batch, node_W, node_b, edge_W, edge_b, lin_W, lin_b, root_emb, bn_g, bn_b, mlp_W1, mlp_b1, mlp_bn1_g, mlp_bn1_b, mlp_W2, mlp_b2, mlp_bn2_g, mlp_bn2_b):
    raise NotImplementedError("write your pallas kernel here")


# speedup vs baseline: 2.7822x; 2.7822x over previous
import functools

import jax
import jax.numpy as jnp
from jax import lax
from jax.experimental import pallas as pl
from jax.experimental.pallas import tpu as pltpu
from jax.experimental.pallas import tpu_sc as plsc

N = 10000
E = 320000
EMB = 128
EDGE_DIM = 16
NUM_GRAPHS = 64
NUM_LAYERS = 3
HID = 64
NUM_CLASSES = 10

NC = 2
NS = 16
NW = NC * NS
CHUNK = 128
N_PAD = 10112
ROWS_PER_TILE = N_PAD // NS
E_PAD = 323584
EW = E_PAD // NW
NCHUNK = EW // CHUNK

_HIGH = jax.lax.Precision.HIGHEST
_DEF = jax.lax.Precision.DEFAULT

_sc_mesh = plsc.VectorSubcoreMesh(
    core_axis_name="c", subcore_axis_name="s", num_cores=NC, num_subcores=NS)
_sc_params = pltpu.CompilerParams(needs_layout_passes=False)


@functools.partial(
    pl.kernel,
    out_type=jax.ShapeDtypeStruct((NC, N_PAD, EMB), jnp.float32),
    mesh=_sc_mesh,
    compiler_params=_sc_params,
    scratch_types=[
        pltpu.VMEM((CHUNK,), jnp.int32),
        pltpu.VMEM((CHUNK, EMB), jnp.float32),
        pltpu.VMEM_SHARED((N_PAD, EMB), jnp.float32),
    ],
)
def _hist_kernel(row_hbm, out_hbm, idx_v, one_v, h_sh):
    c = lax.axis_index("c")
    s = lax.axis_index("s")
    wid = c * NS + s
    base_row = s * ROWS_PER_TILE

    unit = jnp.where(lax.iota(jnp.int32, 16) == 0, 1.0, 0.0).astype(jnp.float32)
    zero16 = jnp.zeros((16,), jnp.float32)

    def zbody(i, _):
        for d in range(EMB // 16):
            one_v[i, pl.ds(d * 16, 16)] = zero16
        return 0

    lax.fori_loop(0, CHUNK, zbody, 0)
    for kk in range(ROWS_PER_TILE // CHUNK):
        pltpu.sync_copy(one_v, h_sh.at[pl.ds(base_row + kk * CHUNK, CHUNK)])
    rem = ROWS_PER_TILE % CHUNK
    if rem:
        pltpu.sync_copy(
            one_v.at[pl.ds(0, rem)],
            h_sh.at[pl.ds(base_row + (ROWS_PER_TILE // CHUNK) * CHUNK, rem)])

    def obody2(i, _):
        one_v[i, pl.ds(0, 16)] = unit
        return 0

    lax.fori_loop(0, CHUNK, obody2, 0)
    plsc.subcore_barrier()

    ebase = wid * EW

    def body(k, _):
        pltpu.sync_copy(row_hbm.at[pl.ds(ebase + k * CHUNK, CHUNK)], idx_v)
        pltpu.sync_copy(one_v, h_sh.at[idx_v], add=True)
        return 0

    lax.fori_loop(0, NCHUNK, body, 0)
    plsc.subcore_barrier()
    for kk in range(ROWS_PER_TILE // CHUNK):
        pltpu.sync_copy(h_sh.at[pl.ds(base_row + kk * CHUNK, CHUNK)], one_v)
        pltpu.sync_copy(one_v,
                        out_hbm.at[c, pl.ds(base_row + kk * CHUNK, CHUNK)])
    if rem:
        tail = base_row + (ROWS_PER_TILE // CHUNK) * CHUNK
        pltpu.sync_copy(h_sh.at[pl.ds(tail, rem)], one_v.at[pl.ds(0, rem)])
        pltpu.sync_copy(one_v.at[pl.ds(0, rem)], out_hbm.at[c, pl.ds(tail, rem)])


@functools.partial(
    pl.kernel,
    out_type=jax.ShapeDtypeStruct((NC, N_PAD, EMB), jnp.float32),
    mesh=_sc_mesh,
    compiler_params=_sc_params,
    scratch_types=[
        pltpu.VMEM((N_PAD,), jnp.float32),
        pltpu.VMEM((CHUNK,), jnp.int32),
        pltpu.VMEM((CHUNK,), jnp.int32),
        pltpu.VMEM((CHUNK,), jnp.float32),
        pltpu.VMEM((CHUNK, EMB), jnp.float32),
        pltpu.VMEM((CHUNK, EMB), jnp.float32),
        pltpu.VMEM_SHARED((N_PAD, EMB), jnp.float32),
        pltpu.SemaphoreType.DMA,
    ],
)
def _edge_kernel(xl_hbm, row_hbm, col_hbm, ea_hbm, dis_hbm, out_hbm,
                 dis_t, row_v, col_v, norm_v, g_v, ea_v, aggr_sh, sem):
    c = lax.axis_index("c")
    s = lax.axis_index("s")
    wid = c * NS + s
    base_row = s * ROWS_PER_TILE

    def zrow(i, _):
        for d in range(EMB // 16):
            g_v[i, pl.ds(d * 16, 16)] = jnp.zeros((16,), jnp.float32)
        return 0

    lax.fori_loop(0, CHUNK, zrow, 0)
    for kk in range(ROWS_PER_TILE // CHUNK):
        pltpu.sync_copy(g_v, aggr_sh.at[pl.ds(base_row + kk * CHUNK, CHUNK)])
    rem = ROWS_PER_TILE % CHUNK
    if rem:
        pltpu.sync_copy(
            g_v.at[pl.ds(0, rem)],
            aggr_sh.at[pl.ds(base_row + (ROWS_PER_TILE // CHUNK) * CHUNK, rem)])
    pltpu.sync_copy(dis_hbm, dis_t)
    plsc.subcore_barrier()

    ebase = wid * EW

    def chunk_body(k, _):
        off = ebase + k * CHUNK
        pltpu.sync_copy(row_hbm.at[pl.ds(off, CHUNK)], row_v)
        pltpu.sync_copy(col_hbm.at[pl.ds(off, CHUNK)], col_v)
        pltpu.async_copy(xl_hbm.at[row_v], g_v, sem).wait()
        pltpu.sync_copy(ea_hbm.at[pl.ds(off, CHUNK)], ea_v)
        for eg in range(CHUNK // 16):
            sl = pl.ds(eg * 16, 16)
            nr = plsc.load_gather(dis_t, [row_v[sl]])
            nc_ = plsc.load_gather(dis_t, [col_v[sl]])
            norm_v[sl] = nr * nc_

        def ebody(e, _):
            idx = jnp.broadcast_to(e, (16,)).astype(jnp.int32)
            nrm = plsc.load_gather(norm_v, [idx])
            for d in range(EMB // 16):
                dsl = pl.ds(d * 16, 16)
                v = g_v[e, dsl] + ea_v[e, dsl]
                g_v[e, dsl] = jnp.maximum(v, 0.0) * nrm
            return 0

        lax.fori_loop(0, CHUNK, ebody, 0)
        pltpu.sync_copy(g_v, aggr_sh.at[col_v], add=True)
        return 0

    lax.fori_loop(0, NCHUNK, chunk_body, 0)
    plsc.subcore_barrier()
    for kk in range(ROWS_PER_TILE // CHUNK):
        pltpu.sync_copy(aggr_sh.at[pl.ds(base_row + kk * CHUNK, CHUNK)], g_v)
        pltpu.sync_copy(g_v, out_hbm.at[c, pl.ds(base_row + kk * CHUNK, CHUNK)])
    if rem:
        tail = base_row + (ROWS_PER_TILE // CHUNK) * CHUNK
        pltpu.sync_copy(aggr_sh.at[pl.ds(tail, rem)], g_v.at[pl.ds(0, rem)])
        pltpu.sync_copy(g_v.at[pl.ds(0, rem)], out_hbm.at[c, pl.ds(tail, rem)])


def _ea_body(attr_ref, w_ref, b_ref, out_ref):
    out_ref[...] = (
        jnp.dot(attr_ref[...], w_ref[...], preferred_element_type=jnp.float32,
                precision=_DEF) + b_ref[...])


_EA_BLK = 1024


def _ea_call(attr_p, edge_W, edge_b):
    grid = (E_PAD // _EA_BLK,)
    return pl.pallas_call(
        _ea_body,
        grid=grid,
        in_specs=[
            pl.BlockSpec((_EA_BLK, EDGE_DIM), lambda i: (i, 0)),
            pl.BlockSpec((EDGE_DIM, EMB), lambda i: (0, 0)),
            pl.BlockSpec((1, EMB), lambda i: (0, 0)),
        ],
        out_specs=pl.BlockSpec((_EA_BLK, EMB), lambda i: (i, 0)),
        out_shape=jax.ShapeDtypeStruct((E_PAD, EMB), jnp.float32),
    )(attr_p, edge_W, edge_b.reshape(1, EMB))


def _prep_body(x_ref, nw_ref, nb_ref, lw_ref, lb_ref, hist_ref,
               h0_ref, xl0_ref, dis_ref, invdeg_ref):
    h0 = jnp.dot(x_ref[...], nw_ref[...], preferred_element_type=jnp.float32,
                 precision=_DEF) + nb_ref[...]
    h0_ref[...] = h0
    xl0_ref[...] = jnp.dot(h0, lw_ref[...], preferred_element_type=jnp.float32,
                           precision=_DEF) + lb_ref[...]
    deg = jnp.sum(hist_ref[0] + hist_ref[1], axis=1) + 1.0
    dis_ref[...] = lax.rsqrt(deg)[None, :]
    invdeg_ref[...] = (1.0 / deg)[None, :]


def _prep_call(x_p, node_W, node_b, lin_W0, lin_b0, hist):
    return pl.pallas_call(
        _prep_body,
        out_shape=[
            jax.ShapeDtypeStruct((N_PAD, EMB), jnp.float32),
            jax.ShapeDtypeStruct((N_PAD, EMB), jnp.float32),
            jax.ShapeDtypeStruct((1, N_PAD), jnp.float32),
            jax.ShapeDtypeStruct((1, N_PAD), jnp.float32),
        ],
    )(x_p, node_W, node_b.reshape(1, EMB), lin_W0, lin_b0.reshape(1, EMB),
      hist)


def _post_body(h_ref, xl_ref, aggr_ref, root_ref, g_ref, b_ref, invdeg_ref,
               w_ref, wb_ref, hn_ref, xln_ref, *, do_relu):
    aggr = aggr_ref[0] + aggr_ref[1]
    self_msg = jax.nn.relu(xl_ref[...] + root_ref[...])
    t = aggr + self_msg * invdeg_ref[0, :][:, None]
    mask = (lax.broadcasted_iota(jnp.int32, (N_PAD, 1), 0) < N).astype(
        jnp.float32)
    mu = jnp.sum(t * mask, axis=0, keepdims=True) / N
    d = t - mu
    var = jnp.sum(d * d * mask, axis=0, keepdims=True) / N
    hn = d * lax.rsqrt(var + 1e-5) * g_ref[...] + b_ref[...]
    if do_relu:
        hn = jax.nn.relu(hn)
    hnew = hn + h_ref[...]
    hn_ref[...] = hnew
    xln_ref[...] = jnp.dot(hnew, w_ref[...], preferred_element_type=jnp.float32,
                           precision=_DEF) + wb_ref[...]


def _post_call(h, xl, aggr, root, g, b, invdeg, w_next, b_next):
    return pl.pallas_call(
        functools.partial(_post_body, do_relu=True),
        out_shape=[
            jax.ShapeDtypeStruct((N_PAD, EMB), jnp.float32),
            jax.ShapeDtypeStruct((N_PAD, EMB), jnp.float32),
        ],
    )(h, xl, aggr, root.reshape(1, EMB), g.reshape(1, EMB), b.reshape(1, EMB),
      invdeg, w_next, b_next.reshape(1, EMB))


def _final_body(h_ref, xl_ref, aggr_ref, root_ref, g_ref, b_ref, invdeg_ref,
                batch_ref, w1_ref, b1_ref, g1_ref, bb1_ref, w2_ref, b2_ref,
                g2_ref, bb2_ref, out_ref):
    aggr = aggr_ref[0] + aggr_ref[1]
    self_msg = jax.nn.relu(xl_ref[...] + root_ref[...])
    t = aggr + self_msg * invdeg_ref[0, :][:, None]
    mask = (lax.broadcasted_iota(jnp.int32, (N_PAD, 1), 0) < N).astype(
        jnp.float32)
    mu = jnp.sum(t * mask, axis=0, keepdims=True) / N
    d = t - mu
    var = jnp.sum(d * d * mask, axis=0, keepdims=True) / N
    hn = d * lax.rsqrt(var + 1e-5) * g_ref[...] + b_ref[...]
    h3 = hn + h_ref[...]

    onehot = (lax.broadcasted_iota(jnp.int32, (NUM_GRAPHS, N_PAD), 0)
              == jnp.broadcast_to(batch_ref[...], (NUM_GRAPHS, N_PAD))
              ).astype(jnp.float32)
    counts = jnp.sum(onehot, axis=1, keepdims=True)
    hsum = jnp.dot(onehot, h3, preferred_element_type=jnp.float32,
                   precision=_HIGH)
    h_graph = hsum / jnp.maximum(counts, 1.0)

    z = jnp.dot(h_graph, w1_ref[...], preferred_element_type=jnp.float32,
                precision=_DEF) + b1_ref[...]
    mu1 = jnp.mean(z, axis=0, keepdims=True)
    v1 = jnp.mean((z - mu1) ** 2, axis=0, keepdims=True)
    z = (z - mu1) * lax.rsqrt(v1 + 1e-5) * g1_ref[...] + bb1_ref[...]
    z = jax.nn.relu(z)
    z = jnp.dot(z, w2_ref[...], preferred_element_type=jnp.float32,
                precision=_DEF) + b2_ref[...]
    mu2 = jnp.mean(z, axis=0, keepdims=True)
    v2 = jnp.mean((z - mu2) ** 2, axis=0, keepdims=True)
    out_ref[...] = (z - mu2) * lax.rsqrt(v2 + 1e-5) * g2_ref[...] + bb2_ref[...]


def _final_call(h, xl, aggr, root, g, b, invdeg, batch_p, mlp_W1, mlp_b1,
                g1, bb1, mlp_W2, mlp_b2, g2, bb2):
    return pl.pallas_call(
        _final_body,
        out_shape=jax.ShapeDtypeStruct((NUM_GRAPHS, NUM_CLASSES), jnp.float32),
    )(h, xl, aggr, root.reshape(1, EMB), g.reshape(1, EMB), b.reshape(1, EMB),
      invdeg, batch_p.reshape(1, N_PAD), mlp_W1, mlp_b1.reshape(1, HID),
      g1.reshape(1, HID), bb1.reshape(1, HID), mlp_W2,
      mlp_b2.reshape(1, NUM_CLASSES), g2.reshape(1, NUM_CLASSES),
      bb2.reshape(1, NUM_CLASSES))


def kernel(x, edge_index, edge_attr, batch, node_W, node_b, edge_W, edge_b,
           lin_W, lin_b, root_emb, bn_g, bn_b, mlp_W1, mlp_b1, mlp_bn1_g,
           mlp_bn1_b, mlp_W2, mlp_b2, mlp_bn2_g, mlp_bn2_b):
    pad_e = E_PAD - E
    pad_n = N_PAD - N
    row_p = jnp.concatenate(
        [edge_index[0], jnp.full((pad_e,), N, jnp.int32)])
    col_p = jnp.concatenate(
        [edge_index[1], jnp.full((pad_e,), N, jnp.int32)])
    attr_p = jnp.pad(edge_attr, ((0, pad_e), (0, 0)))
    x_p = jnp.pad(x, ((0, pad_n), (0, 0)))
    batch_p = jnp.concatenate(
        [batch, jnp.full((pad_n,), NUM_GRAPHS, jnp.int32)])

    hist = _hist_kernel(row_p)
    ea = _ea_call(attr_p, edge_W, edge_b)
    h, xl, dis, invdeg = _prep_call(x_p, node_W, node_b, lin_W[0], lin_b[0],
                                    hist)
    dis_flat = dis.reshape(N_PAD)

    for l in range(NUM_LAYERS - 1):
        aggr = _edge_kernel(xl, row_p, col_p, ea, dis_flat)
        h, xl = _post_call(h, xl, aggr, root_emb[l], bn_g[l], bn_b[l], invdeg,
                           lin_W[l + 1], lin_b[l + 1])

    aggr = _edge_kernel(xl, row_p, col_p, ea, dis_flat)
    z = _final_call(h, xl, aggr, root_emb[2], bn_g[2], bn_b[2], invdeg,
                    batch_p, mlp_W1, mlp_b1, mlp_bn1_g, mlp_bn1_b, mlp_W2,
                    mlp_b2, mlp_bn2_g, mlp_bn2_b)
    return z
